# Initial kernel scaffold; baseline (speedup 1.0000x reference)
import jax
import jax.numpy as jnp
from jax.experimental import pallas as pl


def kernel(x, params):
    raise NotImplementedError("write your pallas kernel here")


# speedup vs baseline: 12.6525x; 12.6525x over previous
import functools

import jax
import jax.numpy as jnp
from jax.experimental import pallas as pl

BS = 32
C = 16
H = 512
W = 16
NUM_LAYERS = 4
K = 8
HEADS = 4
HID = C * W
N = BS * H

_HIGHEST = jax.lax.Precision.HIGHEST


def _adj_kernel(xn_ref, adj_ref):
    xb = xn_ref[0]
    nrm = jnp.sqrt(jnp.sum(xb * xb, axis=1, keepdims=True))
    xnb = xb / (nrm + 1e-12)
    sim = jax.lax.dot_general(
        xnb, xnb, (((1,), (1,)), ((), ())),
        precision=_HIGHEST, preferred_element_type=jnp.float32)
    row = jax.lax.broadcasted_iota(jnp.int32, (H, H), 0)
    col = jax.lax.broadcasted_iota(jnp.int32, (H, H), 1)
    work = jnp.where(row == col, sim - 1e9, sim)
    adj = jnp.zeros((H, H), jnp.float32)
    for _ in range(K):
        m = jnp.max(work, axis=1, keepdims=True)
        is_max = work == m
        cand = jnp.where(is_max, col, H)
        first_idx = jnp.min(cand, axis=1, keepdims=True)
        first = col == first_idx
        adj = adj + first.astype(jnp.float32)
        work = jnp.where(first, -3.0e38, work)
    adj_ref[0] = adj


def _layer_a_kernel(h_ref, adj_ref, wgt_ref, bg_ref, bc_blk_ref, bc_ref,
                    y_ref, stats_ref):
    b = pl.program_id(0)
    h = h_ref[0]
    adj = adj_ref[0]
    xw = jax.lax.dot_general(
        h, wgt_ref[...], (((1,), (0,)), ((), ())),
        precision=_HIGHEST, preferred_element_type=jnp.float32)
    agg = jax.lax.dot_general(
        adj, xw, (((1,), (0,)), ((), ())),
        precision=_HIGHEST, preferred_element_type=jnp.float32)
    agg = (agg + xw) * (1.0 / 9.0) + bg_ref[...]
    r = jnp.maximum(agg, 0.0)
    y = jax.lax.dot_general(
        r, bc_blk_ref[...], (((1,), (0,)), ((), ())),
        precision=_HIGHEST, preferred_element_type=jnp.float32) + bc_ref[...]
    y_ref[0] = y

    @pl.when(b == 0)
    def _():
        stats_ref[...] = jnp.zeros_like(stats_ref)

    stats_ref[0:1, :] += jnp.sum(y, axis=0, keepdims=True)
    stats_ref[1:2, :] += jnp.sum(y * y, axis=0, keepdims=True)


def _layer_b_kernel(y_ref, stats_ref, gm_ref, bt_ref, res_ref, wrt_ref,
                    br_ref, out_ref):
    y = y_ref[0]
    m = stats_ref[0:1, :] * (1.0 / N)
    ex2 = stats_ref[1:2, :] * (1.0 / N)
    v = ex2 - m * m
    yb = gm_ref[...] * (y - m) / jnp.sqrt(v + 1e-5) + bt_ref[...]
    resn = jax.lax.dot_general(
        res_ref[0], wrt_ref[...], (((1,), (0,)), ((), ())),
        precision=_HIGHEST, preferred_element_type=jnp.float32) + br_ref[...]
    out_ref[0] = yb + resn


def _full(shape):
    return pl.BlockSpec(shape, lambda b: (0,) * len(shape))


def _batched(f):
    return pl.BlockSpec((1, H, f), lambda b: (b, 0, 0))


@jax.jit
def kernel(x, params):
    xn = jnp.transpose(x, (0, 2, 1, 3)).reshape(BS, H, C * W)

    adj = pl.pallas_call(
        _adj_kernel,
        grid=(BS,),
        in_specs=[_batched(HID)],
        out_specs=_batched(H),
        out_shape=jax.ShapeDtypeStruct((BS, H, H), jnp.float32),
    )(xn)

    h = xn
    res = xn
    for p in params:
        fo, fi = p['Wg'].shape
        g = fo // HEADS
        wc = p['Wc'].reshape(g, HEADS, HEADS)
        bc_blk = jnp.einsum('gh,goj->gjho', jnp.eye(g, dtype=jnp.float32),
                            wc).reshape(fo, fo)
        y, stats = pl.pallas_call(
            _layer_a_kernel,
            grid=(BS,),
            in_specs=[_batched(fi), _batched(H), _full((fi, fo)),
                      _full((1, fo)), _full((fo, fo)), _full((1, fo))],
            out_specs=[_batched(fo), _full((2, fo))],
            out_shape=[jax.ShapeDtypeStruct((BS, H, fo), jnp.float32),
                       jax.ShapeDtypeStruct((2, fo), jnp.float32)],
        )(h, adj, p['Wg'].T, p['bg'][None, :], bc_blk, p['bc'][None, :])

        h = pl.pallas_call(
            _layer_b_kernel,
            grid=(BS,),
            in_specs=[_batched(fo), _full((2, fo)), _full((1, fo)),
                      _full((1, fo)), _batched(fi), _full((fi, fo)),
                      _full((1, fo))],
            out_specs=_batched(fo),
            out_shape=jax.ShapeDtypeStruct((BS, H, fo), jnp.float32),
        )(y, stats, p['gm'][None, :], p['bt'][None, :], res, p['Wr'].T,
          p['br'][None, :])
        res = h

    return h.reshape(BS, H * (HID >> NUM_LAYERS))


# speedup vs baseline: 15.7762x; 1.2469x over previous
import jax
import jax.numpy as jnp
from jax.experimental import pallas as pl

BS = 32
C = 16
H = 512
W = 16
NUM_LAYERS = 4
K = 8
HEADS = 4
HID = C * W
N = BS * H

_HIGHEST = jax.lax.Precision.HIGHEST


def _dot(a, b, precision=None):
    return jax.lax.dot_general(a, b, (((1,), (0,)), ((), ())),
                               precision=precision,
                               preferred_element_type=jnp.float32)


def _split(a):
    bits = jax.lax.bitcast_convert_type(a, jnp.uint32)
    hi_f32 = jax.lax.bitcast_convert_type(
        bits & jnp.uint32(0xFFFF0000), jnp.float32)
    hi = hi_f32.astype(jnp.bfloat16)
    lo = (a - hi_f32).astype(jnp.bfloat16)
    return hi, lo


def _dot3(a, b):
    ah, al = _split(a)
    bh, bl = _split(b)
    return _dot(ah, bh) + (_dot(ah, bl) + _dot(al, bh))


def _adj_matmul(adj16, xw):
    return _dot(adj16.astype(jnp.float32), xw, _HIGHEST)


def _build_adj(xb):
    nrm = jnp.sqrt(jnp.sum(xb * xb, axis=1, keepdims=True))
    xnb = xb / (nrm + 1e-12)
    sim = jax.lax.dot_general(
        xnb, xnb, (((1,), (1,)), ((), ())),
        precision=_HIGHEST, preferred_element_type=jnp.float32)
    row = jax.lax.broadcasted_iota(jnp.int32, (H, H), 0)
    col = jax.lax.broadcasted_iota(jnp.int32, (H, H), 1)
    work = jnp.where(row == col, sim - 1e9, sim)
    adj = jnp.zeros((H, H), jnp.float32)
    for _ in range(K):
        m = jnp.max(work, axis=1, keepdims=True)
        is_max = work == m
        cand = jnp.where(is_max, col, H)
        first_idx = jnp.min(cand, axis=1, keepdims=True)
        first = col == first_idx
        adj = adj + first.astype(jnp.float32)
        work = jnp.where(first, -3.0e38, work)
    return adj


def _layer_a(h, adj16, wgt_ref, bg_ref, bc_blk_ref, bc_ref, b,
             y_ref, stats_ref):
    xw = _dot3(h, wgt_ref[...])
    agg = (_adj_matmul(adj16, xw) + xw) * (1.0 / 9.0) + bg_ref[...]
    r = jnp.maximum(agg, 0.0)
    y = _dot3(r, bc_blk_ref[...]) + bc_ref[...]
    y_ref[0] = y

    @pl.when(b == 0)
    def _():
        stats_ref[...] = jnp.zeros_like(stats_ref)

    stats_ref[0:1, :] += jnp.sum(y, axis=0, keepdims=True)
    stats_ref[1:2, :] += jnp.sum(y * y, axis=0, keepdims=True)


def _layer_b(y_ref, stats_ref, gm_ref, bt_ref, res, wrt_ref, br_ref):
    y = y_ref[0]
    m = stats_ref[0:1, :] * (1.0 / N)
    ex2 = stats_ref[1:2, :] * (1.0 / N)
    v = ex2 - m * m
    yb = gm_ref[...] * (y - m) / jnp.sqrt(v + 1e-5) + bt_ref[...]
    resn = _dot3(res, wrt_ref[...]) + br_ref[...]
    return yb + resn


def _p0_kernel(xn_ref, wgt_ref, bg_ref, bc_blk_ref, bc_ref,
               adj_ref, y_ref, stats_ref):
    b = pl.program_id(0)
    adj = _build_adj(xn_ref[0])
    adj16 = adj.astype(jnp.bfloat16)
    adj_ref[0] = adj16
    _layer_a(xn_ref[0], adj16, wgt_ref, bg_ref, bc_blk_ref, bc_ref, b,
             y_ref, stats_ref)


def _pmid_kernel(y_ref, stats_ref, gm_ref, bt_ref, res_ref, wrt_ref, br_ref,
                 adj_ref, wgt_ref, bg_ref, bc_blk_ref, bc_ref,
                 h_ref, y2_ref, stats2_ref):
    b = pl.program_id(0)
    h = _layer_b(y_ref, stats_ref, gm_ref, bt_ref, res_ref[0], wrt_ref,
                 br_ref)
    h_ref[0] = h
    _layer_a(h, adj_ref[0], wgt_ref, bg_ref, bc_blk_ref, bc_ref, b,
             y2_ref, stats2_ref)


def _plast_kernel(y_ref, stats_ref, gm_ref, bt_ref, res_ref, wrt_ref, br_ref,
                  h_ref):
    h_ref[0] = _layer_b(y_ref, stats_ref, gm_ref, bt_ref, res_ref[0],
                        wrt_ref, br_ref)


def _full(shape):
    return pl.BlockSpec(shape, lambda b: (0,) * len(shape))


def _batched(f):
    return pl.BlockSpec((1, H, f), lambda b: (b, 0, 0))


def _layer_a_args(p):
    fo, fi = p['Wg'].shape
    g = fo // HEADS
    wc = p['Wc'].reshape(g, HEADS, HEADS)
    bc_blk = jnp.einsum('gh,goj->gjho', jnp.eye(g, dtype=jnp.float32),
                        wc).reshape(fo, fo)
    specs = [_full((fi, fo)), _full((1, fo)), _full((fo, fo)),
             _full((1, fo))]
    vals = (p['Wg'].T, p['bg'][None, :], bc_blk, p['bc'][None, :])
    return specs, vals


def _layer_b_args(p):
    fo, fi = p['Wg'].shape
    specs = [_full((2, fo)), _full((1, fo)), _full((1, fo)),
             _batched(fi), _full((fi, fo)), _full((1, fo))]

    def vals(stats, res):
        return (stats, p['gm'][None, :], p['bt'][None, :], res,
                p['Wr'].T, p['br'][None, :])

    return specs, vals


def _ab_out(fo):
    return ([_batched(fo), _full((2, fo))],
            [jax.ShapeDtypeStruct((BS, H, fo), jnp.float32),
             jax.ShapeDtypeStruct((2, fo), jnp.float32)])


@jax.jit
def kernel(x, params):
    xn = jnp.transpose(x, (0, 2, 1, 3)).reshape(BS, H, C * W)
    fos = [p['Wg'].shape[0] for p in params]

    a_specs0, a_vals0 = _layer_a_args(params[0])
    out_specs, out_shape = _ab_out(fos[0])
    adj, y, stats = pl.pallas_call(
        _p0_kernel,
        grid=(BS,),
        in_specs=[_batched(HID)] + a_specs0,
        out_specs=[_batched(H)] + out_specs,
        out_shape=[jax.ShapeDtypeStruct((BS, H, H), jnp.bfloat16)]
        + out_shape,
    )(xn, *a_vals0)

    res = xn
    for i in range(NUM_LAYERS - 1):
        b_specs, b_vals = _layer_b_args(params[i])
        a_specs, a_vals = _layer_a_args(params[i + 1])
        out_specs, out_shape = _ab_out(fos[i + 1])
        h, y, stats = pl.pallas_call(
            _pmid_kernel,
            grid=(BS,),
            in_specs=[_batched(fos[i])] + b_specs + [_batched(H)] + a_specs,
            out_specs=[_batched(fos[i])] + out_specs,
            out_shape=[jax.ShapeDtypeStruct((BS, H, fos[i]), jnp.float32)]
            + out_shape,
        )(y, *b_vals(stats, res), adj, *a_vals)
        res = h

    b_specs, b_vals = _layer_b_args(params[-1])
    h = pl.pallas_call(
        _plast_kernel,
        grid=(BS,),
        in_specs=[_batched(fos[-1])] + b_specs,
        out_specs=_batched(fos[-1]),
        out_shape=jax.ShapeDtypeStruct((BS, H, fos[-1]), jnp.float32),
    )(y, *b_vals(stats, res))

    return h.reshape(BS, H * (HID >> NUM_LAYERS))
